# baseline (speedup 1.0000x reference)
import jax
import jax.numpy as jnp
from jax.experimental import pallas as pl
from jax.experimental.pallas import tpu as pltpu


def _copy_kernel(x_ref, o_ref):
    C = x_ref.shape[2]
    o_ref[:, :, 0:C] = x_ref[...]
    o_ref[:, :, C:2 * C] = x_ref[...]


def kernel(x, weight, bias):
    T, N, C, H, W = x.shape
    M = 2 * C
    out = pl.pallas_call(
        _copy_kernel,
        out_shape=jax.ShapeDtypeStruct((T, N, M, H, W), x.dtype),
        grid=(T * 4,),
        in_specs=[pl.BlockSpec((1, N // 4, C, H, W),
                               lambda i: (i // 4, i % 4, 0, 0, 0))],
        out_specs=pl.BlockSpec((1, N // 4, M, H, W),
                               lambda i: (i // 4, i % 4, 0, 0, 0)),
        compiler_params=pltpu.CompilerParams(
            dimension_semantics=("parallel",),
            vmem_limit_bytes=48 * 1024 * 1024,
        ),
    )(x)
    return out


# speedup vs baseline: 2.2342x; 2.2342x over previous
import functools

import jax
import jax.numpy as jnp
from jax.experimental import pallas as pl
from jax.experimental.pallas import tpu as pltpu


def _conv_bd_kernel(x_ref, w_ref, b_ref, o_ref, xpad_ref, xcol0_ref,
                    xcol1_ref, *, H, W, K, M, C, bt, pad_lanes):
    HW = H * W
    p = K // 2
    f32 = jnp.float32

    zeros_halo = jnp.zeros((C, pad_lanes), xpad_ref.dtype)
    xpad_ref[:, 0:pad_lanes] = zeros_halo
    xpad_ref[:, pad_lanes + HW:2 * pad_lanes + HW] = zeros_halo

    col = jax.lax.broadcasted_iota(jnp.int32, (1, HW), 1) % W
    col_masks = []
    for dx in range(K):
        dxo = dx - p
        if dxo == 0:
            col_masks.append(None)
        else:
            col_masks.append(
                ((col + dxo >= 0) & (col + dxo < W)).astype(xpad_ref.dtype))

    bias = b_ref[...]
    xcols = (xcol0_ref, xcol1_ref)
    for b in range(bt):
        xcol_ref = xcols[b % 2]
        xpad_ref[:, pad_lanes:pad_lanes + HW] = x_ref[b]
        for dy in range(K):
            for dx in range(K):
                t = dy * K + dx
                s = (dy - p) * W + (dx - p)
                win = xpad_ref[:, pad_lanes + s:pad_lanes + s + HW]
                if col_masks[dx] is not None:
                    win = win * col_masks[dx]
                xcol_ref[t * C:(t + 1) * C, :] = win
        acc = jax.lax.dot_general(
            w_ref[...], xcol_ref[...],
            (((1,), (0,)), ((), ())),
            preferred_element_type=f32)
        o_ref[b] = (acc + bias).astype(o_ref.dtype)


def _conv_basis(x, weight, bias, basis_size, kernel_size):
    K = kernel_size
    T, N, C, H, W = x.shape
    n_basis = weight.shape[0]
    p = K // 2
    group = C // basis_size
    HW = H * W
    B = T * N
    M = group * n_basis

    pad_lanes = 128 * ((p * W + p + 127) // 128)

    wt = jnp.transpose(weight, (2, 3, 0, 1)).reshape(K * K, n_basis,
                                                     basis_size)
    eye = jnp.eye(group, dtype=weight.dtype)
    w_bd = jnp.einsum('gh,tnc->tgnhc', eye, wt).reshape(K * K, M, C)
    w2 = jnp.transpose(w_bd, (1, 0, 2)).reshape(M, K * K * C).astype(
        jnp.bfloat16)
    b_bd = jnp.tile(bias, group).reshape(M, 1).astype(jnp.float32)

    xr = x.reshape(B, C, HW).astype(jnp.bfloat16)

    bt = 4
    while B % bt != 0:
        bt //= 2

    kfn = functools.partial(_conv_bd_kernel, H=H, W=W, K=K, M=M, C=C,
                            bt=bt, pad_lanes=pad_lanes)

    out = pl.pallas_call(
        kfn,
        out_shape=jax.ShapeDtypeStruct((B, M, HW), jnp.bfloat16),
        grid=(B // bt,),
        in_specs=[
            pl.BlockSpec((bt, C, HW), lambda i: (i, 0, 0)),
            pl.BlockSpec((M, K * K * C), lambda i: (0, 0)),
            pl.BlockSpec((M, 1), lambda i: (0, 0)),
        ],
        out_specs=pl.BlockSpec((bt, M, HW), lambda i: (i, 0, 0)),
        scratch_shapes=[
            pltpu.VMEM((C, HW + 2 * pad_lanes), jnp.bfloat16),
            pltpu.VMEM((K * K * C, HW), jnp.bfloat16),
            pltpu.VMEM((K * K * C, HW), jnp.bfloat16),
        ],
        compiler_params=pltpu.CompilerParams(
            dimension_semantics=("parallel",),
            vmem_limit_bytes=48 * 1024 * 1024,
        ),
    )(xr, w2, b_bd)

    return out.astype(x.dtype).reshape(T, N, M, H, W)


def kernel(x, weight, bias):
    return _conv_basis(x, weight, bias, 4, 3)


# speedup vs baseline: 3.2614x; 1.4598x over previous
import functools

import jax
import jax.numpy as jnp
from jax.experimental import pallas as pl
from jax.experimental.pallas import tpu as pltpu


def _conv_bd_kernel(x_ref, w_ref, b_ref, o_ref, xpad_ref, xcol0_ref,
                    xcol1_ref, *, H, W, K, M, C, bt, pad_rows):
    HW = H * W
    p = K // 2
    f32 = jnp.float32

    zeros_halo = jnp.zeros((pad_rows, C), xpad_ref.dtype)
    xpad_ref[0:pad_rows, :] = zeros_halo
    xpad_ref[pad_rows + HW:2 * pad_rows + HW, :] = zeros_halo

    row = jax.lax.broadcasted_iota(jnp.int32, (HW, 1), 0) % W
    row_masks = []
    for dx in range(K):
        dxo = dx - p
        if dxo == 0:
            row_masks.append(None)
        else:
            row_masks.append(
                ((row + dxo >= 0) & (row + dxo < W)).astype(xpad_ref.dtype))

    bias = b_ref[...]
    xcols = (xcol0_ref, xcol1_ref)
    for b in range(bt):
        xcol_ref = xcols[b % 2]
        xpad_ref[pad_rows:pad_rows + HW, :] = x_ref[b].astype(xpad_ref.dtype)
        for dy in range(K):
            for dx in range(K):
                t = dy * K + dx
                s = (dy - p) * W + (dx - p)
                win = xpad_ref[pl.ds(pad_rows + s, HW), :]
                if row_masks[dx] is not None:
                    win = win * row_masks[dx]
                xcol_ref[:, t * C:(t + 1) * C] = win
        acc = jax.lax.dot_general(
            xcol_ref[...], w_ref[...],
            (((1,), (0,)), ((), ())),
            preferred_element_type=f32)
        o_ref[b] = (acc + bias).astype(o_ref.dtype)


def _conv_basis(x, weight, bias, basis_size, kernel_size):
    K = kernel_size
    T, N, C, H, W = x.shape
    n_basis = weight.shape[0]
    p = K // 2
    group = C // basis_size
    HW = H * W
    B = T * N
    M = group * n_basis

    pad_rows = 8 * ((p * W + p + 7) // 8)

    wt = jnp.transpose(weight, (2, 3, 1, 0)).reshape(K * K, basis_size,
                                                     n_basis)
    eye = jnp.eye(group, dtype=weight.dtype)
    w_bd = jnp.einsum('gh,tcn->tgchn', eye, wt).reshape(
        K * K * C, group * n_basis).astype(jnp.bfloat16)
    b_bd = jnp.tile(bias, group).reshape(1, M).astype(jnp.float32)

    xv = jnp.transpose(x, (0, 1, 3, 4, 2)).reshape(B, HW, C)

    bt = 4
    while B % bt != 0:
        bt //= 2

    kfn = functools.partial(_conv_bd_kernel, H=H, W=W, K=K, M=M, C=C,
                            bt=bt, pad_rows=pad_rows)

    out = pl.pallas_call(
        kfn,
        out_shape=jax.ShapeDtypeStruct((B, HW, M), x.dtype),
        grid=(B // bt,),
        in_specs=[
            pl.BlockSpec((bt, HW, C), lambda i: (i, 0, 0)),
            pl.BlockSpec((K * K * C, M), lambda i: (0, 0)),
            pl.BlockSpec((1, M), lambda i: (0, 0)),
        ],
        out_specs=pl.BlockSpec((bt, HW, M), lambda i: (i, 0, 0)),
        scratch_shapes=[
            pltpu.VMEM((HW + 2 * pad_rows, C), jnp.bfloat16),
            pltpu.VMEM((HW, K * K * C), jnp.bfloat16),
            pltpu.VMEM((HW, K * K * C), jnp.bfloat16),
        ],
        compiler_params=pltpu.CompilerParams(
            dimension_semantics=("parallel",),
            vmem_limit_bytes=48 * 1024 * 1024,
        ),
    )(xv, w_bd, b_bd)

    return jnp.transpose(out.reshape(T, N, H, W, M), (0, 1, 4, 2, 3))


def kernel(x, weight, bias):
    return _conv_basis(x, weight, bias, 4, 3)


# speedup vs baseline: 3.8085x; 1.1678x over previous
import functools

import jax
import jax.numpy as jnp
from jax.experimental import pallas as pl
from jax.experimental.pallas import tpu as pltpu


def _conv_bd_kernel(x_ref, w_ref, b_ref, o_ref, xpad_ref, xcol0_ref,
                    xcol1_ref, *, H, W, K, M, C, bt, pad_rows):
    HW = H * W
    p = K // 2
    f32 = jnp.float32

    zeros_halo = jnp.zeros((pad_rows, C), xpad_ref.dtype)
    xpad_ref[0:pad_rows, :] = zeros_halo
    xpad_ref[pad_rows + HW:2 * pad_rows + HW, :] = zeros_halo

    row = jax.lax.broadcasted_iota(jnp.int32, (HW, 1), 0) % W
    row_masks = []
    for dx in range(K):
        dxo = dx - p
        if dxo == 0:
            row_masks.append(None)
        else:
            row_masks.append(
                ((row + dxo >= 0) & (row + dxo < W)).astype(xpad_ref.dtype))

    bias = b_ref[...]
    xcols = (xcol0_ref, xcol1_ref)
    for b in range(bt):
        xcol_ref = xcols[b % 2]
        xpad_ref[pad_rows:pad_rows + HW, :] = x_ref[b].astype(xpad_ref.dtype)
        for dy in range(K):
            for dx in range(K):
                t = dy * K + dx
                s = (dy - p) * W + (dx - p)
                win = xpad_ref[pl.ds(pad_rows + s, HW), :]
                if row_masks[dx] is not None:
                    win = win * row_masks[dx]
                xcol_ref[:, t * C:(t + 1) * C] = win
        acc = jax.lax.dot_general(
            xcol_ref[...], w_ref[...],
            (((1,), (0,)), ((), ())),
            preferred_element_type=f32)
        o_ref[b] = (acc + bias).astype(o_ref.dtype)


def _conv_basis(x, weight, bias, basis_size, kernel_size):
    K = kernel_size
    T, N, C, H, W = x.shape
    n_basis = weight.shape[0]
    p = K // 2
    group = C // basis_size
    HW = H * W
    B = T * N
    M = group * n_basis

    pad_rows = 8 * ((p * W + p + 7) // 8)

    KKC = K * K * C
    wt = jnp.transpose(weight, (2, 3, 1, 0)).reshape(K * K, 1, basis_size,
                                                     n_basis)
    w_cols = jnp.tile(jnp.broadcast_to(wt, (K * K, group, basis_size,
                                            n_basis)).reshape(KKC, n_basis),
                      (1, group))
    rows = jax.lax.broadcasted_iota(jnp.int32, (KKC, M), 0)
    cols = jax.lax.broadcasted_iota(jnp.int32, (KKC, M), 1)
    keep = ((rows % C) // basis_size) == (cols // n_basis)
    w2 = jnp.where(keep, w_cols, 0.0).astype(jnp.bfloat16)
    b_bd = jnp.tile(bias, group).reshape(1, M).astype(jnp.float32)

    xv = jnp.transpose(x, (0, 1, 3, 4, 2)).reshape(B, HW, C)

    bt = 4
    while B % bt != 0:
        bt //= 2

    kfn = functools.partial(_conv_bd_kernel, H=H, W=W, K=K, M=M, C=C,
                            bt=bt, pad_rows=pad_rows)

    out = pl.pallas_call(
        kfn,
        out_shape=jax.ShapeDtypeStruct((B, HW, M), x.dtype),
        grid=(B // bt,),
        in_specs=[
            pl.BlockSpec((bt, HW, C), lambda i: (i, 0, 0)),
            pl.BlockSpec((K * K * C, M), lambda i: (0, 0)),
            pl.BlockSpec((1, M), lambda i: (0, 0)),
        ],
        out_specs=pl.BlockSpec((bt, HW, M), lambda i: (i, 0, 0)),
        scratch_shapes=[
            pltpu.VMEM((HW + 2 * pad_rows, C), jnp.bfloat16),
            pltpu.VMEM((HW, K * K * C), jnp.bfloat16),
            pltpu.VMEM((HW, K * K * C), jnp.bfloat16),
        ],
        compiler_params=pltpu.CompilerParams(
            dimension_semantics=("parallel",),
            vmem_limit_bytes=48 * 1024 * 1024,
        ),
    )(xv, w2, b_bd)

    return jnp.transpose(out.reshape(T, N, H, W, M), (0, 1, 4, 2, 3))


def kernel(x, weight, bias):
    return _conv_basis(x, weight, bias, 4, 3)


# speedup vs baseline: 3.8382x; 1.0078x over previous
import functools

import jax
import jax.numpy as jnp
from jax.experimental import pallas as pl
from jax.experimental.pallas import tpu as pltpu


def _conv_bd_kernel(x_ref, w_ref, b_ref, o_ref, xpad_ref, xcol0_ref,
                    xcol1_ref, *, H, W, K, M, C, bt, pad_rows):
    HW = H * W
    p = K // 2
    f32 = jnp.float32

    zeros_halo = jnp.zeros((pad_rows, C), xpad_ref.dtype)
    xpad_ref[0:pad_rows, :] = zeros_halo
    xpad_ref[pad_rows + HW:2 * pad_rows + HW, :] = zeros_halo

    row = jax.lax.broadcasted_iota(jnp.int32, (HW, 1), 0) % W
    row_masks = []
    for dx in range(K):
        dxo = dx - p
        if dxo == 0:
            row_masks.append(None)
        else:
            row_masks.append(
                ((row + dxo >= 0) & (row + dxo < W)).astype(xpad_ref.dtype))

    bias = b_ref[...]
    xcols = (xcol0_ref, xcol1_ref)
    for b in range(bt):
        xcol_ref = xcols[b % 2]
        xpad_ref[pad_rows:pad_rows + HW, :] = x_ref[b].astype(xpad_ref.dtype)
        for dy in range(K):
            for dx in range(K):
                t = dy * K + dx
                s = (dy - p) * W + (dx - p)
                win = xpad_ref[pl.ds(pad_rows + s, HW), :]
                if row_masks[dx] is not None:
                    win = win * row_masks[dx]
                xcol_ref[:, t * C:(t + 1) * C] = win
        acc = jax.lax.dot_general(
            xcol_ref[...], w_ref[...],
            (((1,), (0,)), ((), ())),
            preferred_element_type=f32)
        o_ref[b] = (acc + bias).astype(o_ref.dtype)


def _conv_basis(x, weight, bias, basis_size, kernel_size):
    K = kernel_size
    T, N, C, H, W = x.shape
    n_basis = weight.shape[0]
    p = K // 2
    group = C // basis_size
    HW = H * W
    B = T * N
    M = group * n_basis

    pad_rows = 8 * ((p * W + p + 7) // 8)

    KKC = K * K * C
    wt = jnp.transpose(weight, (2, 3, 1, 0)).reshape(K * K, 1, basis_size,
                                                     n_basis)
    w_cols = jnp.tile(jnp.broadcast_to(wt, (K * K, group, basis_size,
                                            n_basis)).reshape(KKC, n_basis),
                      (1, group))
    rows = jax.lax.broadcasted_iota(jnp.int32, (KKC, M), 0)
    cols = jax.lax.broadcasted_iota(jnp.int32, (KKC, M), 1)
    keep = ((rows % C) // basis_size) == (cols // n_basis)
    w2 = jnp.where(keep, w_cols, 0.0).astype(jnp.bfloat16)
    b_bd = jnp.tile(bias, group).reshape(1, M).astype(jnp.float32)

    xv = jnp.transpose(x, (0, 1, 3, 4, 2)).reshape(B, HW, C)

    bt = 8
    while B % bt != 0:
        bt //= 2

    kfn = functools.partial(_conv_bd_kernel, H=H, W=W, K=K, M=M, C=C,
                            bt=bt, pad_rows=pad_rows)

    out = pl.pallas_call(
        kfn,
        out_shape=jax.ShapeDtypeStruct((B, HW, M), x.dtype),
        grid=(B // bt,),
        in_specs=[
            pl.BlockSpec((bt, HW, C), lambda i: (i, 0, 0)),
            pl.BlockSpec((K * K * C, M), lambda i: (0, 0)),
            pl.BlockSpec((1, M), lambda i: (0, 0)),
        ],
        out_specs=pl.BlockSpec((bt, HW, M), lambda i: (i, 0, 0)),
        scratch_shapes=[
            pltpu.VMEM((HW + 2 * pad_rows, C), jnp.bfloat16),
            pltpu.VMEM((HW, K * K * C), jnp.bfloat16),
            pltpu.VMEM((HW, K * K * C), jnp.bfloat16),
        ],
        compiler_params=pltpu.CompilerParams(
            dimension_semantics=("parallel",),
            vmem_limit_bytes=48 * 1024 * 1024,
        ),
    )(xv, w2, b_bd)

    return jnp.transpose(out.reshape(T, N, H, W, M), (0, 1, 4, 2, 3))


def kernel(x, weight, bias):
    return _conv_basis(x, weight, bias, 4, 3)


# speedup vs baseline: 4.9016x; 1.2771x over previous
import functools

import jax
import jax.numpy as jnp
from jax.experimental import pallas as pl
from jax.experimental.pallas import tpu as pltpu


def _conv_bd_kernel(x_ref, w_ref, b_ref, o_ref, xpad_ref, xshm_ref, xshp_ref,
                    xcol0_ref, xcol1_ref, *, H, W, K, M, C, bt, pad_rows):
    HW = H * W
    p = K // 2
    f32 = jnp.float32
    nsh = HW + 2 * W

    zeros_halo = jnp.zeros((pad_rows, C), xpad_ref.dtype)
    xpad_ref[0:pad_rows, :] = zeros_halo
    xpad_ref[pad_rows + HW:2 * pad_rows + HW, :] = zeros_halo

    row = jax.lax.broadcasted_iota(jnp.int32, (nsh, 1), 0) % W
    mask_m = (row >= 1).astype(xpad_ref.dtype)
    mask_p = (row <= W - 2).astype(xpad_ref.dtype)

    bias = b_ref[...]
    xcols = (xcol0_ref, xcol1_ref)
    for b in range(bt):
        xcol_ref = xcols[b % 2]
        xpad_ref[pad_rows:pad_rows + HW, :] = x_ref[b].astype(xpad_ref.dtype)
        xshm_ref[...] = xpad_ref[pl.ds(pad_rows - W - 1, nsh), :] * mask_m
        xshp_ref[...] = xpad_ref[pl.ds(pad_rows - W + 1, nsh), :] * mask_p
        for dy in range(K):
            dyo = dy - p
            for dx in range(K):
                t = dy * K + dx
                if dx == p:
                    win = xpad_ref[pl.ds(pad_rows + dyo * W, HW), :]
                elif dx < p:
                    win = xshm_ref[pl.ds(W + dyo * W, HW), :]
                else:
                    win = xshp_ref[pl.ds(W + dyo * W, HW), :]
                xcol_ref[:, t * C:(t + 1) * C] = win
        acc = jax.lax.dot_general(
            xcol_ref[...], w_ref[...],
            (((1,), (0,)), ((), ())),
            preferred_element_type=f32)
        o_ref[b] = (acc + bias).astype(o_ref.dtype)


def _conv_basis(x, weight, bias, basis_size, kernel_size):
    K = kernel_size
    T, N, C, H, W = x.shape
    n_basis = weight.shape[0]
    p = K // 2
    group = C // basis_size
    HW = H * W
    B = T * N
    M = group * n_basis

    pad_rows = 8 * ((p * W + p + 7) // 8)

    KKC = K * K * C
    wt = jnp.transpose(weight, (2, 3, 1, 0)).reshape(K * K, 1, basis_size,
                                                     n_basis)
    w_cols = jnp.tile(jnp.broadcast_to(wt, (K * K, group, basis_size,
                                            n_basis)).reshape(KKC, n_basis),
                      (1, group))
    rows = jax.lax.broadcasted_iota(jnp.int32, (KKC, M), 0)
    cols = jax.lax.broadcasted_iota(jnp.int32, (KKC, M), 1)
    keep = ((rows % C) // basis_size) == (cols // n_basis)
    w2 = jnp.where(keep, w_cols, 0.0).astype(jnp.bfloat16)
    b_bd = jnp.tile(bias, group).reshape(1, M).astype(jnp.float32)

    xv = jnp.transpose(x, (0, 1, 3, 4, 2)).reshape(B, HW, C)

    bt = 8
    while B % bt != 0:
        bt //= 2

    kfn = functools.partial(_conv_bd_kernel, H=H, W=W, K=K, M=M, C=C,
                            bt=bt, pad_rows=pad_rows)

    out = pl.pallas_call(
        kfn,
        out_shape=jax.ShapeDtypeStruct((B, HW, M), x.dtype),
        grid=(B // bt,),
        in_specs=[
            pl.BlockSpec((bt, HW, C), lambda i: (i, 0, 0)),
            pl.BlockSpec((K * K * C, M), lambda i: (0, 0)),
            pl.BlockSpec((1, M), lambda i: (0, 0)),
        ],
        out_specs=pl.BlockSpec((bt, HW, M), lambda i: (i, 0, 0)),
        scratch_shapes=[
            pltpu.VMEM((HW + 2 * pad_rows, C), jnp.bfloat16),
            pltpu.VMEM((HW + 2 * W, C), jnp.bfloat16),
            pltpu.VMEM((HW + 2 * W, C), jnp.bfloat16),
            pltpu.VMEM((HW, K * K * C), jnp.bfloat16),
            pltpu.VMEM((HW, K * K * C), jnp.bfloat16),
        ],
        compiler_params=pltpu.CompilerParams(
            dimension_semantics=("parallel",),
            vmem_limit_bytes=48 * 1024 * 1024,
        ),
    )(xv, w2, b_bd)

    return jnp.transpose(out.reshape(T, N, H, W, M), (0, 1, 4, 2, 3))


def kernel(x, weight, bias):
    return _conv_basis(x, weight, bias, 4, 3)
